# baseline (speedup 1.0000x reference)
import jax
import jax.numpy as jnp
from jax.experimental import pallas as pl
from jax.experimental.pallas import tpu as pltpu

_INSTANCE_OFFSET = 1000
_NUM_THINGS = 80
_NUM_STUFF = 53
_IGNORE = 53
_STUFF_AREA_THR = 4096
_THING_CONF_THR = 0.5


def _fusion_body(gind_ref, score_ref, cls_ref, mask_ref, sem_ref, out_ref,
                 insid_ref, pan_ref, avail_ref):
    del gind_ref
    b = pl.program_id(0)
    i = pl.program_id(1)
    n = pl.num_programs(1)

    @pl.when(i == 0)
    def _init():
        pan_ref[...] = jnp.zeros(pan_ref.shape, pan_ref.dtype)
        avail_ref[...] = jnp.ones(avail_ref.shape, avail_ref.dtype)
        insid_ref[0] = jnp.int32(1)

    ones_row = jnp.ones((1, mask_ref.shape[2]), jnp.bfloat16)

    def _count(mbf):
        cols = jax.lax.dot_general(
            ones_row, mbf, (((1,), (0,)), ((), ())),
            preferred_element_type=jnp.float32)
        return jnp.sum(cols)

    @pl.when(score_ref[b, i] >= _THING_CONF_THR)
    def _instance():
        maskb = mask_ref[0, 0].astype(jnp.bfloat16)
        avail = avail_ref[...]
        freeb = maskb * avail
        mask_area = _count(maskb)
        free_area = _count(freeb)
        inter_area = mask_area - free_area
        keep = jnp.logical_and(mask_area > 0.0,
                               2.0 * inter_area <= mask_area)

        @pl.when(keep)
        def _paint():
            ins_id = insid_ref[0]
            label = cls_ref[b, i] + ins_id * _INSTANCE_OFFSET
            pan_ref[...] = jnp.where(freeb != 0.0, label, pan_ref[...])
            avail_ref[...] = avail - freeb
            insid_ref[0] = ins_id + 1

    @pl.when(i == n - 1)
    def _stuff():
        covered = avail_ref[...] == 0.0
        pan = pan_ref[...]
        semb = jnp.where(covered, jnp.bfloat16(_IGNORE), sem_ref[0])
        lo = jnp.int32(0)
        hi = jnp.int32(0)
        for c in range(_NUM_STUFF):
            cnt = _count((semb == c).astype(jnp.bfloat16))
            ok = (cnt >= _STUFF_AREA_THR).astype(jnp.int32)
            if c < 32:
                lo = lo + (ok << c)
            else:
                hi = hi + (ok << (c - 32))
        sem = semb.astype(jnp.int32)
        word = jnp.where(sem < 32, lo, hi)
        shift = jnp.where(sem < 32, sem, sem - 32)
        okpix = ((word >> shift) & 1) == 1
        out_ref[0] = jnp.where(covered, pan,
                               jnp.where(okpix, sem + _NUM_THINGS, 0))


def kernel(ins_masks_masks, ins_masks_scores, ins_masks_class_ids, sem_masks):
    B, N, H, W = ins_masks_masks.shape
    sorted_inds = jnp.argsort(-ins_masks_scores, axis=1).astype(jnp.int32)
    s_scores = jnp.take_along_axis(ins_masks_scores, sorted_inds, axis=1)
    s_cls = jnp.take_along_axis(
        ins_masks_class_ids.astype(jnp.int32), sorted_inds, axis=1)
    k = jnp.sum((s_scores >= _THING_CONF_THR).astype(jnp.int32), axis=1)
    eff = jnp.minimum(jnp.arange(N, dtype=jnp.int32)[None, :],
                      jnp.maximum(k[:, None] - 1, 0))
    g_inds = jnp.take_along_axis(sorted_inds, eff, axis=1)

    grid_spec = pltpu.PrefetchScalarGridSpec(
        num_scalar_prefetch=3,
        grid=(B, N),
        in_specs=[
            pl.BlockSpec((1, 1, H, W),
                         lambda b, i, gind, sc, cl: (b, gind[b, i], 0, 0)),
            pl.BlockSpec((1, H, W), lambda b, i, gind, sc, cl: (b, 0, 0)),
        ],
        out_specs=pl.BlockSpec((1, H, W), lambda b, i, gind, sc, cl: (b, 0, 0)),
        scratch_shapes=[
            pltpu.SMEM((1,), jnp.int32),
            pltpu.VMEM((H, W), jnp.int32),
            pltpu.VMEM((H, W), jnp.bfloat16),
        ],
    )
    return pl.pallas_call(
        _fusion_body,
        grid_spec=grid_spec,
        out_shape=jax.ShapeDtypeStruct((B, H, W), jnp.int32),
    )(g_inds, s_scores, s_cls, ins_masks_masks,
      sem_masks.astype(jnp.bfloat16))


# speedup vs baseline: 1.4960x; 1.4960x over previous
import jax
import jax.numpy as jnp
from jax.experimental import pallas as pl
from jax.experimental.pallas import tpu as pltpu

_INSTANCE_OFFSET = 1000
_NUM_THINGS = 80
_NUM_STUFF = 53
_IGNORE = 53
_STUFF_AREA_THR = 4096
_THING_CONF_THR = 0.5


def _fusion_body(gind_ref, score_ref, cls_ref, mask_ref, sem_ref, out_ref,
                 insid_ref, pan_ref, avail_ref):
    del gind_ref
    b = pl.program_id(0)
    i = pl.program_id(1)
    n = pl.num_programs(1)

    @pl.when(i == 0)
    def _init():
        pan_ref[...] = jnp.zeros(pan_ref.shape, pan_ref.dtype)
        avail_ref[...] = jnp.ones(avail_ref.shape, avail_ref.dtype)
        insid_ref[0] = jnp.int32(1)

    ones_row = jnp.ones((1, mask_ref.shape[2]), jnp.bfloat16)

    def _count(mbf):
        cols = jax.lax.dot_general(
            ones_row, mbf, (((1,), (0,)), ((), ())),
            preferred_element_type=jnp.float32)
        return jnp.sum(cols)

    @pl.when(score_ref[b, i] >= _THING_CONF_THR)
    def _instance():
        maskb = mask_ref[0, 0].astype(jnp.bfloat16)
        avail = avail_ref[...]
        freeb = maskb * avail
        mask_area = _count(maskb)
        free_area = _count(freeb)
        inter_area = mask_area - free_area
        keep = jnp.logical_and(mask_area > 0.0,
                               2.0 * inter_area <= mask_area)

        @pl.when(keep)
        def _paint():
            ins_id = insid_ref[0]
            label = cls_ref[b, i] + ins_id * _INSTANCE_OFFSET
            pan_ref[...] = jnp.where(freeb != 0.0, label, pan_ref[...])
            avail_ref[...] = avail - freeb
            insid_ref[0] = ins_id + 1

    @pl.when(i == n - 1)
    def _stuff():
        covered = avail_ref[...] == 0.0
        pan = pan_ref[...]
        semb = jnp.where(covered, jnp.bfloat16(_IGNORE), sem_ref[0])
        lo = jnp.int32(0)
        hi = jnp.int32(0)
        for c in range(_NUM_STUFF):
            cnt = _count((semb == c).astype(jnp.bfloat16))
            ok = (cnt >= _STUFF_AREA_THR).astype(jnp.int32)
            if c < 32:
                lo = lo + (ok << c)
            else:
                hi = hi + (ok << (c - 32))
        sem = semb.astype(jnp.int32)
        word = jnp.where(sem < 32, lo, hi)
        shift = jnp.where(sem < 32, sem, sem - 32)
        okpix = ((word >> shift) & 1) == 1
        out_ref[0] = jnp.where(covered, pan,
                               jnp.where(okpix, sem + _NUM_THINGS, 0))


def kernel(ins_masks_masks, ins_masks_scores, ins_masks_class_ids, sem_masks):
    B, N, H, W = ins_masks_masks.shape
    sorted_inds = jnp.argsort(-ins_masks_scores, axis=1).astype(jnp.int32)
    s_scores = jnp.take_along_axis(ins_masks_scores, sorted_inds, axis=1)
    s_cls = jnp.take_along_axis(
        ins_masks_class_ids.astype(jnp.int32), sorted_inds, axis=1)
    k = jnp.sum((s_scores >= _THING_CONF_THR).astype(jnp.int32), axis=1)
    eff = jnp.minimum(jnp.arange(N, dtype=jnp.int32)[None, :],
                      jnp.maximum(k[:, None] - 1, 0))
    g_inds = jnp.take_along_axis(sorted_inds, eff, axis=1)

    grid_spec = pltpu.PrefetchScalarGridSpec(
        num_scalar_prefetch=3,
        grid=(B, N),
        in_specs=[
            pl.BlockSpec((1, 1, H, W),
                         lambda b, i, gind, sc, cl: (b, gind[b, i], 0, 0)),
            pl.BlockSpec((1, H, W), lambda b, i, gind, sc, cl: (b, 0, 0)),
        ],
        out_specs=pl.BlockSpec((1, H, W), lambda b, i, gind, sc, cl: (b, 0, 0)),
        scratch_shapes=[
            pltpu.SMEM((1,), jnp.int32),
            pltpu.VMEM((H, W), jnp.int32),
            pltpu.VMEM((H, W), jnp.bfloat16),
        ],
    )
    return pl.pallas_call(
        _fusion_body,
        grid_spec=grid_spec,
        out_shape=jax.ShapeDtypeStruct((B, H, W), jnp.int32),
    )(g_inds, s_scores, s_cls, ins_masks_masks.view(jnp.int8),
      sem_masks.astype(jnp.bfloat16))


# speedup vs baseline: 2.2800x; 1.5240x over previous
import jax
import jax.numpy as jnp
from jax.experimental import pallas as pl
from jax.experimental.pallas import tpu as pltpu

_INSTANCE_OFFSET = 1000
_NUM_THINGS = 80
_NUM_STUFF = 53
_IGNORE = 53
_STUFF_AREA_THR = 4096
_THING_CONF_THR = 0.5


_PACK = 8


def _fusion_body(gind_ref, score_ref, cls_ref, m0_ref, m1_ref, m2_ref, m3_ref,
                 m4_ref, m5_ref, m6_ref, m7_ref,
                 sem_ref, out_ref, insid_ref, pan_ref, avail_ref):
    del gind_ref
    mask_refs = (m0_ref, m1_ref, m2_ref, m3_ref, m4_ref, m5_ref, m6_ref,
                 m7_ref)
    b = pl.program_id(0)
    i = pl.program_id(1)
    n = pl.num_programs(1)

    @pl.when(i == 0)
    def _init():
        pan_ref[...] = jnp.zeros(pan_ref.shape, pan_ref.dtype)
        avail_ref[...] = jnp.ones(avail_ref.shape, avail_ref.dtype)
        insid_ref[0] = jnp.int32(1)

    ones_row = jnp.ones((1, m0_ref.shape[2]), jnp.bfloat16)

    def _count(mbf):
        cols = jax.lax.dot_general(
            ones_row, mbf, (((1,), (0,)), ((), ())),
            preferred_element_type=jnp.float32)
        return jnp.sum(cols)

    def _keep(mask_area, free_area):
        return jnp.logical_and(mask_area > 0.0,
                               2.0 * (mask_area - free_area) <= mask_area)

    def _paint(freeb, ci):
        ins_id = insid_ref[0]
        label = cls_ref[b, ci] + ins_id * _INSTANCE_OFFSET
        pan_ref[...] = jnp.where(freeb != 0.0, label, pan_ref[...])
        avail_ref[...] = avail_ref[...] - freeb
        insid_ref[0] = ins_id + 1

    @pl.when(score_ref[b, _PACK * i] >= _THING_CONF_THR)
    def _inst_group():
        mbs = [r[0, 0].astype(jnp.bfloat16) for r in mask_refs]
        a = avail_ref[...]
        fspecs = [m * a for m in mbs]
        areas = [_count(m) for m in mbs]
        fareas = [_count(f) for f in fspecs]
        start = insid_ref[0]

        @pl.when(_keep(areas[0], fareas[0]))
        def _p0():
            _paint(fspecs[0], _PACK * i)

        def _inst_j(j):
            @pl.when(insid_ref[0] == start)
            def _clean():
                @pl.when(_keep(areas[j], fareas[j]))
                def _pj():
                    _paint(fspecs[j], _PACK * i + j)

            @pl.when(insid_ref[0] != start)
            def _dirty():
                fj = mbs[j] * avail_ref[...]

                @pl.when(_keep(areas[j], _count(fj)))
                def _pj():
                    _paint(fj, _PACK * i + j)

        def _gate(j):
            @pl.when(score_ref[b, _PACK * i + j] >= _THING_CONF_THR)
            def _g():
                _inst_j(j)
                if j + 1 < _PACK:
                    _gate(j + 1)

        _gate(1)

    @pl.when(i == n - 1)
    def _stuff():
        covered = avail_ref[...] == 0.0
        pan = pan_ref[...]
        sem = jnp.where(covered, jnp.int32(_IGNORE), sem_ref[0])
        lo = jnp.int32(0)
        hi = jnp.int32(0)
        for c in range(_NUM_STUFF):
            cnt = _count((sem == c).astype(jnp.bfloat16))
            ok = (cnt >= _STUFF_AREA_THR).astype(jnp.int32)
            if c < 32:
                lo = lo + (ok << c)
            else:
                hi = hi + (ok << (c - 32))
        word = jnp.where(sem < 32, lo, hi)
        shift = jnp.where(sem < 32, sem, sem - 32)
        okpix = ((word >> shift) & 1) == 1
        out_ref[0] = jnp.where(covered, pan,
                               jnp.where(okpix, sem + _NUM_THINGS, 0))


def kernel(ins_masks_masks, ins_masks_scores, ins_masks_class_ids, sem_masks):
    B, N, H, W = ins_masks_masks.shape
    sorted_inds = jnp.argsort(-ins_masks_scores, axis=1).astype(jnp.int32)
    s_scores = jnp.take_along_axis(ins_masks_scores, sorted_inds, axis=1)
    s_cls = jnp.take_along_axis(
        ins_masks_class_ids.astype(jnp.int32), sorted_inds, axis=1)
    k = jnp.sum((s_scores >= _THING_CONF_THR).astype(jnp.int32), axis=1)
    eff = jnp.minimum(jnp.arange(N, dtype=jnp.int32)[None, :],
                      jnp.maximum(k[:, None] - 1, 0))
    g_inds = jnp.take_along_axis(sorted_inds, eff, axis=1)

    Np = ((N + _PACK - 1) // _PACK) * _PACK
    if Np != N:
        pad = Np - N
        s_scores = jnp.pad(s_scores, ((0, 0), (0, pad)),
                           constant_values=-1.0)
        s_cls = jnp.pad(s_cls, ((0, 0), (0, pad)))
        g_inds = jnp.concatenate(
            [g_inds, jnp.repeat(g_inds[:, -1:], pad, axis=1)], axis=1)

    def _mask_spec(j):
        return pl.BlockSpec(
            (1, 1, H, W),
            lambda b, i, gind, sc, cl: (b, gind[b, _PACK * i + j], 0, 0))

    grid_spec = pltpu.PrefetchScalarGridSpec(
        num_scalar_prefetch=3,
        grid=(B, Np // _PACK),
        in_specs=[_mask_spec(j) for j in range(_PACK)] + [
            pl.BlockSpec((1, H, W), lambda b, i, gind, sc, cl: (b, 0, 0)),
        ],
        out_specs=pl.BlockSpec((1, H, W), lambda b, i, gind, sc, cl: (b, 0, 0)),
        scratch_shapes=[
            pltpu.SMEM((1,), jnp.int32),
            pltpu.VMEM((H, W), jnp.int32),
            pltpu.VMEM((H, W), jnp.bfloat16),
        ],
    )
    masks8 = ins_masks_masks.view(jnp.int8)
    return pl.pallas_call(
        _fusion_body,
        grid_spec=grid_spec,
        out_shape=jax.ShapeDtypeStruct((B, H, W), jnp.int32),
    )(g_inds, s_scores, s_cls, *([masks8] * _PACK),
      sem_masks.astype(jnp.int32))
